# Initial kernel scaffold; baseline (speedup 1.0000x reference)
import jax
import jax.numpy as jnp
from jax.experimental import pallas as pl


def kernel(feat, edge_index, edge_feat, W_fc, attn_l, attn_r, W_edge, attn_edge, bias, W_res):
    raise NotImplementedError("write your pallas kernel here")


# speedup vs baseline: 2.1795x; 2.1795x over previous
import jax
import jax.numpy as jnp
from jax import lax
from jax.experimental import pallas as pl
from jax.experimental.pallas import tpu as pltpu

_CP = pltpu.CompilerParams(vmem_limit_bytes=32 * 1024 * 1024)


def kernel(feat, edge_index, edge_feat, W_fc, attn_l, attn_r, W_edge, attn_edge, bias, W_res):
    f32 = jnp.float32
    i32 = jnp.int32
    N, IN = feat.shape
    E, EF = edge_feat.shape
    H, D = attn_l.shape[1], attn_l.shape[2]
    HD = H * D
    HE = H * EF

    src = edge_index[0].astype(i32)
    dst = edge_index[1].astype(i32)

    W3 = W_fc.reshape(H, D, IN)
    P = jnp.concatenate([
        jnp.einsum('hd,hdi->hi', attn_l[0], W3),
        jnp.einsum('hd,hdi->hi', attn_r[0], W3)], axis=0)
    We3 = W_edge.reshape(H, D, EF)
    Q = jnp.einsum('hd,hdf->hf', attn_edge[0], We3)
    W_big = jnp.zeros((HE, HD), f32)
    for h in range(H):
        W_big = W_big.at[h * EF:(h + 1) * EF, h * D:(h + 1) * D].set(We3[h].T)

    dn = (((1,), (1,)), ((), ()))
    Nb = 2000
    Eb = 512
    EBLK = E // Eb
    NBLK = N // Nb

    def node_mm(x_ref, wfc_ref, wres_ref, p_ref, fs_ref, res_ref, elr_ref):
        x = x_ref[...]
        fs_ref[...] = lax.dot_general(x, wfc_ref[...], dn, preferred_element_type=f32)
        res_ref[...] = lax.dot_general(x, wres_ref[...], dn, preferred_element_type=f32)
        elr_ref[...] = lax.dot_general(x, p_ref[...], dn, preferred_element_type=f32)

    Nb1 = 1000
    fs, res, elr = pl.pallas_call(
        node_mm,
        grid=(N // Nb1,),
        in_specs=[pl.BlockSpec((Nb1, IN), lambda i: (i, 0)),
                  pl.BlockSpec((HD, IN), lambda i: (0, 0)),
                  pl.BlockSpec((HD, IN), lambda i: (0, 0)),
                  pl.BlockSpec((2 * H, IN), lambda i: (0, 0))],
        out_specs=[pl.BlockSpec((Nb1, HD), lambda i: (i, 0)),
                   pl.BlockSpec((Nb1, HD), lambda i: (i, 0)),
                   pl.BlockSpec((Nb1, 2 * H), lambda i: (i, 0))],
        out_shape=[jax.ShapeDtypeStruct((N, HD), f32),
                   jax.ShapeDtypeStruct((N, HD), f32),
                   jax.ShapeDtypeStruct((N, 2 * H), f32)],
        compiler_params=_CP,
    )(feat, W_fc, W_res, P)

    def edge_mm(ef_ref, q_ref, ee_ref):
        ee_ref[...] = lax.dot_general(ef_ref[...], q_ref[...], dn, preferred_element_type=f32)

    Eb1 = 8000
    ee = pl.pallas_call(
        edge_mm,
        grid=(E // Eb1,),
        in_specs=[pl.BlockSpec((Eb1, EF), lambda i: (i, 0)),
                  pl.BlockSpec((H, EF), lambda i: (0, 0))],
        out_specs=pl.BlockSpec((Eb1, H), lambda i: (i, 0)),
        out_shape=jax.ShapeDtypeStruct((E, H), f32),
        compiler_params=_CP,
    )(edge_feat, Q)

    def onehot(idx_vec, nblk):
        cols = lax.broadcasted_iota(i32, (Eb, Nb), 1) + nblk * Nb
        return jnp.where(idx_vec.reshape(Eb, 1) == cols, 1.0, 0.0).astype(f32)

    def escore(src_ref, dst_ref, ee_ref, elr_ref, ex_ref, acc):
        n = pl.program_id(1)

        @pl.when(n == 0)
        def _():
            acc[...] = jnp.zeros_like(acc)

        ps = onehot(src_ref[...], n)
        pd = onehot(dst_ref[...], n)
        elb = elr_ref[...]
        acc[...] += (jnp.dot(ps, elb[:, 0:H], preferred_element_type=f32)
                     + jnp.dot(pd, elb[:, H:2 * H], preferred_element_type=f32))

        @pl.when(n == NBLK - 1)
        def _():
            e = acc[...] + ee_ref[...]
            e = jnp.where(e > 0.0, e, 0.2 * e)
            ex_ref[...] = jnp.exp(e)

    ex = pl.pallas_call(
        escore,
        grid=(EBLK, NBLK),
        in_specs=[pl.BlockSpec((Eb,), lambda e, n: (e,)),
                  pl.BlockSpec((Eb,), lambda e, n: (e,)),
                  pl.BlockSpec((Eb, H), lambda e, n: (e, 0)),
                  pl.BlockSpec((Nb, 2 * H), lambda e, n: (n, 0))],
        out_specs=pl.BlockSpec((Eb, H), lambda e, n: (e, 0)),
        out_shape=jax.ShapeDtypeStruct((E, H), f32),
        scratch_shapes=[pltpu.VMEM((Eb, H), f32)],
        compiler_params=_CP,
    )(src, dst, ee, elr)

    def densum(dst_ref, ex_ref, den_ref):
        e = pl.program_id(1)

        @pl.when(e == 0)
        def _():
            den_ref[...] = jnp.zeros_like(den_ref)

        pd = onehot(dst_ref[...], pl.program_id(0))
        den_ref[...] += lax.dot_general(pd, ex_ref[...], (((0,), (0,)), ((), ())),
                                        preferred_element_type=f32)

    den = pl.pallas_call(
        densum,
        grid=(NBLK, EBLK),
        in_specs=[pl.BlockSpec((Eb,), lambda n, e: (e,)),
                  pl.BlockSpec((Eb, H), lambda n, e: (e, 0))],
        out_specs=pl.BlockSpec((Nb, H), lambda n, e: (n, 0)),
        out_shape=jax.ShapeDtypeStruct((N, H), f32),
        compiler_params=_CP,
    )(dst, ex)

    def invd_mm(dp_ref, invd_ref):
        c = dp_ref[...]
        invd_ref[...] = 1.0 / jnp.where(c == 0.0, 1.0, c)

    invd = pl.pallas_call(
        invd_mm,
        grid=(1,),
        in_specs=[pl.BlockSpec((N, H), lambda i: (0, 0))],
        out_specs=pl.BlockSpec((N, H), lambda i: (0, 0)),
        out_shape=jax.ShapeDtypeStruct((N, H), f32),
        compiler_params=_CP,
    )(den)

    def amsg(src_ref, dst_ref, ex_ref, invd_ref, fs_ref, a_ref, msg_ref, aiv, afg):
        n = pl.program_id(1)

        @pl.when(n == 0)
        def _():
            aiv[...] = jnp.zeros_like(aiv)
            afg[...] = jnp.zeros_like(afg)

        ps = onehot(src_ref[...], n)
        pd = onehot(dst_ref[...], n)
        aiv[...] += jnp.dot(pd, invd_ref[...], preferred_element_type=f32)
        afg[...] += jnp.dot(ps, fs_ref[...], preferred_element_type=f32)

        @pl.when(n == NBLK - 1)
        def _():
            a = ex_ref[...] * aiv[...]
            a_ref[...] = a
            fg = afg[...]
            msg_ref[...] = jnp.concatenate(
                [a[:, h:h + 1] * fg[:, h * D:(h + 1) * D] for h in range(H)], axis=1)

    a_arr, msg = pl.pallas_call(
        amsg,
        grid=(EBLK, NBLK),
        in_specs=[pl.BlockSpec((Eb,), lambda e, n: (e,)),
                  pl.BlockSpec((Eb,), lambda e, n: (e,)),
                  pl.BlockSpec((Eb, H), lambda e, n: (e, 0)),
                  pl.BlockSpec((Nb, H), lambda e, n: (n, 0)),
                  pl.BlockSpec((Nb, HD), lambda e, n: (n, 0))],
        out_specs=[pl.BlockSpec((Eb, H), lambda e, n: (e, 0)),
                   pl.BlockSpec((Eb, HD), lambda e, n: (e, 0))],
        out_shape=[jax.ShapeDtypeStruct((E, H), f32),
                   jax.ShapeDtypeStruct((E, HD), f32)],
        scratch_shapes=[pltpu.VMEM((Eb, H), f32), pltpu.VMEM((Eb, HD), f32)],
        compiler_params=_CP,
    )(src, dst, ex, invd, fs)

    def scat(dst_ref, msg_ref, a_ref, ef_ref, rst_ref, s_ref):
        e = pl.program_id(1)

        @pl.when(e == 0)
        def _():
            rst_ref[...] = jnp.zeros_like(rst_ref)
            s_ref[...] = jnp.zeros_like(s_ref)

        pd = onehot(dst_ref[...], pl.program_id(0))
        dt = (((0,), (0,)), ((), ()))
        rst_ref[...] += lax.dot_general(pd, msg_ref[...], dt, preferred_element_type=f32)
        a = a_ref[...]
        ef = ef_ref[...]
        m2 = jnp.concatenate([a[:, h:h + 1] * ef for h in range(H)], axis=1)
        s_ref[...] += lax.dot_general(pd, m2, dt, preferred_element_type=f32)

    rst, S = pl.pallas_call(
        scat,
        grid=(NBLK, EBLK),
        in_specs=[pl.BlockSpec((Eb,), lambda n, e: (e,)),
                  pl.BlockSpec((Eb, HD), lambda n, e: (e, 0)),
                  pl.BlockSpec((Eb, H), lambda n, e: (e, 0)),
                  pl.BlockSpec((Eb, EF), lambda n, e: (e, 0))],
        out_specs=[pl.BlockSpec((Nb, HD), lambda n, e: (n, 0)),
                   pl.BlockSpec((Nb, HE), lambda n, e: (n, 0))],
        out_shape=[jax.ShapeDtypeStruct((N, HD), f32),
                   jax.ShapeDtypeStruct((N, HE), f32)],
        compiler_params=_CP,
    )(dst, msg, a_arr, edge_feat)

    def final_mm(rst_ref, s_ref, res_ref, wbig_ref, bias_ref, out_ref):
        acc = rst_ref[...] + res_ref[...]
        acc = acc + jnp.dot(s_ref[...], wbig_ref[...], preferred_element_type=f32)
        out_ref[...] = acc + bias_ref[...]

    Nb2 = 1000
    out = pl.pallas_call(
        final_mm,
        grid=(N // Nb2,),
        in_specs=[pl.BlockSpec((Nb2, HD), lambda i: (i, 0)),
                  pl.BlockSpec((Nb2, HE), lambda i: (i, 0)),
                  pl.BlockSpec((Nb2, HD), lambda i: (i, 0)),
                  pl.BlockSpec((HE, HD), lambda i: (0, 0)),
                  pl.BlockSpec((1, HD), lambda i: (0, 0))],
        out_specs=pl.BlockSpec((Nb2, HD), lambda i: (i, 0)),
        out_shape=jax.ShapeDtypeStruct((N, HD), f32),
        compiler_params=_CP,
    )(rst, S, res, W_big, bias.reshape(1, HD))

    return out.reshape(N, H, D)
